# baseline (speedup 1.0000x reference)
import functools
import math

import jax
import jax.numpy as jnp
from jax import lax
from jax.experimental import pallas as pl
from jax.experimental.pallas import tpu as pltpu
from jax.experimental.pallas import tpu_sc as plsc

D_MODEL = 64
SCALE = math.sqrt(D_MODEL)
LANES = 16
NBUF = 4


@functools.lru_cache(maxsize=None)
def _build_call(rows: int, seq: int, vocab: int, d: int):
    info = plsc.get_sparse_core_info()
    nc, ns = info.num_cores, info.num_subcores
    nw = nc * ns
    assert rows % (nw * NBUF) == 0 and seq % 8 == 0
    r_per_w = rows // nw
    mesh = plsc.VectorSubcoreMesh(core_axis_name="c", subcore_axis_name="s")

    @functools.partial(
        pl.kernel,
        mesh=mesh,
        out_type=jax.ShapeDtypeStruct((rows, seq, d), jnp.float32),
        scratch_types=[
            pltpu.VMEM((r_per_w, seq), jnp.int32),
            pltpu.VMEM((NBUF, seq, d), jnp.float32),
        ]
        + [pltpu.SemaphoreType.DMA] * (2 * NBUF),
        compiler_params=pltpu.CompilerParams(use_tc_tiling_on_sc=False),
    )
    def emb(table_hbm, x_hbm, out_hbm, idx_v, rows_v, *sems):
        sg, sw = sems[:NBUF], sems[NBUF:]
        wid = lax.axis_index("s") * nc + lax.axis_index("c")
        base = wid * r_per_w
        pltpu.sync_copy(x_hbm.at[pl.ds(base, r_per_w)], idx_v)

        def gather_start(g, b):
            pltpu.async_copy(table_hbm.at[idx_v.at[g]], rows_v.at[b], sg[b])

        for b in range(NBUF - 1):
            gather_start(b, b)

        @pl.loop(0, r_per_w, step=NBUF)
        def outer(gg):
            for b in range(NBUF):
                g = gg + b
                pltpu.make_async_copy(
                    table_hbm.at[idx_v.at[0]], rows_v.at[b], sg[b]
                ).wait()

                @plsc.parallel_loop(0, seq, unroll=4)
                def mul(i):
                    for j in range(d // LANES):
                        sl = pl.ds(j * LANES, LANES)
                        rows_v[b, i, sl] = rows_v[b, i, sl] * SCALE

                pltpu.async_copy(rows_v.at[b], out_hbm.at[base + g], sw[b])

                nxt = g + NBUF - 1
                bf = (b + NBUF - 1) % NBUF

                @pl.when(jnp.logical_and(nxt < r_per_w, g >= 1))
                def _():
                    pltpu.make_async_copy(
                        rows_v.at[bf], out_hbm.at[base], sw[bf]
                    ).wait()
                    gather_start(nxt, bf)

                @pl.when(jnp.logical_and(nxt < r_per_w, g < 1))
                def _():
                    gather_start(nxt, bf)

        for b in range(NBUF):
            pltpu.make_async_copy(rows_v.at[b], out_hbm.at[base], sw[b]).wait()

    return emb


def kernel(x, table):
    vocab, d = table.shape
    rows, seq = x.shape
    return _build_call(rows, seq, vocab, d)(table, x.astype(jnp.int32))
